# Initial kernel scaffold; baseline (speedup 1.0000x reference)
import jax
import jax.numpy as jnp
from jax.experimental import pallas as pl


def kernel(tt_input, gate_w, gate_bias, w_gate, w_up, w_down):
    raise NotImplementedError("write your pallas kernel here")


# speedup vs baseline: 1.7088x; 1.7088x over previous
import jax
import jax.numpy as jnp
from jax.experimental import pallas as pl
from jax.experimental.pallas import tpu as pltpu

_E = 64
_K = 8
_NG = 8
_KG = 4
_GS = _E // _NG
_D = 1024
_F = 512
_T = 128
_SCALE = 2.5


def _gate_weights(x, gw, gb):
    logits = jax.lax.dot_general(x, gw, (((1,), (1,)), ((), ())),
                                 preferred_element_type=jnp.float32)
    scores = jax.nn.sigmoid(logits)
    s4c = scores + gb
    lane = jax.lax.broadcasted_iota(jnp.int32, (_T, _E), 1)
    grp = lane // _GS
    neg = jnp.float32(-jnp.inf)

    gs_cols = []
    for g in range(_NG):
        vg = jnp.where(grp == g, s4c, neg)
        m1 = jnp.max(vg, axis=1, keepdims=True)
        j1 = jnp.min(jnp.where(vg == m1, lane, _E), axis=1, keepdims=True)
        m2 = jnp.max(jnp.where(lane == j1, neg, vg), axis=1, keepdims=True)
        gs_cols.append(m1 + m2)
    gs = jnp.concatenate(gs_cols, axis=1)

    giota = jax.lax.broadcasted_iota(jnp.int32, (_T, _NG), 1)
    gmask = jnp.zeros((_T, _NG), jnp.bool_)
    for _ in range(_KG):
        m = jnp.max(gs, axis=1, keepdims=True)
        jm = jnp.min(jnp.where(gs == m, giota, _NG), axis=1, keepdims=True)
        pick = giota == jm
        gmask = jnp.logical_or(gmask, pick)
        gs = jnp.where(pick, neg, gs)

    emask = jnp.zeros((_T, _E), jnp.bool_)
    for g in range(_NG):
        emask = jnp.logical_or(emask, jnp.logical_and(grp == g, gmask[:, g:g + 1]))

    masked = jnp.where(emask, s4c, neg)
    sel = jnp.zeros((_T, _E), jnp.bool_)
    for _ in range(_K):
        m = jnp.max(masked, axis=1, keepdims=True)
        jm = jnp.min(jnp.where(masked == m, lane, _E), axis=1, keepdims=True)
        pick = lane == jm
        sel = jnp.logical_or(sel, pick)
        masked = jnp.where(pick, neg, masked)

    w = jnp.where(sel, scores, 0.0)
    denom = jnp.sum(w, axis=1, keepdims=True) + 1e-20
    return w * (_SCALE / denom)


def _moe_kernel(x_ref, gw_ref, gb_ref, wg_ref, wu_ref, wd_ref, out_ref, wd_scr):
    e = pl.program_id(0)

    @pl.when(e == 0)
    def _():
        wd_scr[...] = _gate_weights(x_ref[...], gw_ref[...], gb_ref[...])

    xb = x_ref[...].astype(jnp.bfloat16)
    wgb = wg_ref[0].astype(jnp.bfloat16)
    wub = wu_ref[0].astype(jnp.bfloat16)
    wdb = wd_ref[0].astype(jnp.bfloat16)
    h = jnp.dot(xb, wgb, preferred_element_type=jnp.float32)
    u = jnp.dot(xb, wub, preferred_element_type=jnp.float32)
    act = (h * jax.nn.sigmoid(h)) * u
    y = jnp.dot(act.astype(jnp.bfloat16), wdb, preferred_element_type=jnp.float32)

    lane = jax.lax.broadcasted_iota(jnp.int32, (_T, _E), 1)
    wcol = jnp.sum(jnp.where(lane == e, wd_scr[...], 0.0), axis=1, keepdims=True)
    contrib = y * wcol

    @pl.when(e == 0)
    def _():
        out_ref[...] = contrib

    @pl.when(e > 0)
    def _():
        out_ref[...] += contrib


def kernel(tt_input, gate_w, gate_bias, w_gate, w_up, w_down):
    gb2 = gate_bias.reshape(1, _E)
    return pl.pallas_call(
        _moe_kernel,
        grid=(_E,),
        in_specs=[
            pl.BlockSpec((_T, _D), lambda e: (0, 0)),
            pl.BlockSpec((_E, _D), lambda e: (0, 0)),
            pl.BlockSpec((1, _E), lambda e: (0, 0)),
            pl.BlockSpec((1, _D, _F), lambda e: (e, 0, 0)),
            pl.BlockSpec((1, _D, _F), lambda e: (e, 0, 0)),
            pl.BlockSpec((1, _F, _D), lambda e: (e, 0, 0)),
        ],
        out_specs=pl.BlockSpec((_T, _D), lambda e: (0, 0)),
        out_shape=jax.ShapeDtypeStruct((_T, _D), jnp.float32),
        scratch_shapes=[pltpu.VMEM((_T, _E), jnp.float32)],
        compiler_params=pltpu.CompilerParams(
            dimension_semantics=("arbitrary",),
        ),
    )(tt_input, gate_w, gb2, w_gate, w_up, w_down)


# speedup vs baseline: 1.7101x; 1.0008x over previous
import jax
import jax.numpy as jnp
from jax.experimental import pallas as pl
from jax.experimental.pallas import tpu as pltpu

_E = 64
_K = 8
_NG = 8
_KG = 4
_GS = _E // _NG
_D = 1024
_F = 512
_T = 128
_SCALE = 2.5


def _gate_weights(x, gw, gb):
    logits = jax.lax.dot_general(x, gw, (((1,), (1,)), ((), ())),
                                 preferred_element_type=jnp.float32)
    scores = jax.nn.sigmoid(logits)
    s4c = scores + gb
    lane = jax.lax.broadcasted_iota(jnp.int32, (_T, _E), 1)
    grp = lane // _GS
    neg = jnp.float32(-jnp.inf)

    gs_cols = []
    for g in range(_NG):
        vg = jnp.where(grp == g, s4c, neg)
        m1 = jnp.max(vg, axis=1, keepdims=True)
        j1 = jnp.min(jnp.where(vg == m1, lane, _E), axis=1, keepdims=True)
        m2 = jnp.max(jnp.where(lane == j1, neg, vg), axis=1, keepdims=True)
        gs_cols.append(m1 + m2)
    gs = jnp.concatenate(gs_cols, axis=1)

    giota = jax.lax.broadcasted_iota(jnp.int32, (_T, _NG), 1)
    gmask = jnp.zeros((_T, _NG), jnp.bool_)
    for _ in range(_KG):
        m = jnp.max(gs, axis=1, keepdims=True)
        jm = jnp.min(jnp.where(gs == m, giota, _NG), axis=1, keepdims=True)
        pick = giota == jm
        gmask = jnp.logical_or(gmask, pick)
        gs = jnp.where(pick, neg, gs)

    emask = jnp.zeros((_T, _E), jnp.bool_)
    for g in range(_NG):
        emask = jnp.logical_or(emask, jnp.logical_and(grp == g, gmask[:, g:g + 1]))

    masked = jnp.where(emask, s4c, neg)
    sel = jnp.zeros((_T, _E), jnp.bool_)
    for _ in range(_K):
        m = jnp.max(masked, axis=1, keepdims=True)
        jm = jnp.min(jnp.where(masked == m, lane, _E), axis=1, keepdims=True)
        pick = lane == jm
        sel = jnp.logical_or(sel, pick)
        masked = jnp.where(pick, neg, masked)

    w = jnp.where(sel, scores, 0.0)
    denom = jnp.sum(w, axis=1, keepdims=True) + 1e-20
    return w * (_SCALE / denom)


def _moe_kernel(x_ref, gw_ref, gb_ref, wg_ref, wu_ref, wd_ref, out_ref, wd_scr):
    e = pl.program_id(0)

    @pl.when(e == 0)
    def _():
        wd_scr[...] = _gate_weights(x_ref[...], gw_ref[...], gb_ref[...])

    x = x_ref[...]
    h = jnp.dot(x, wg_ref[0], preferred_element_type=jnp.float32)
    u = jnp.dot(x, wu_ref[0], preferred_element_type=jnp.float32)
    act = (h * jax.nn.sigmoid(h)) * u
    y = jnp.dot(act, wd_ref[0], preferred_element_type=jnp.float32)

    lane = jax.lax.broadcasted_iota(jnp.int32, (_T, _E), 1)
    wcol = jnp.sum(jnp.where(lane == e, wd_scr[...], 0.0), axis=1, keepdims=True)
    contrib = y * wcol

    @pl.when(e == 0)
    def _():
        out_ref[...] = contrib

    @pl.when(e > 0)
    def _():
        out_ref[...] += contrib


def kernel(tt_input, gate_w, gate_bias, w_gate, w_up, w_down):
    gb2 = gate_bias.reshape(1, _E)
    return pl.pallas_call(
        _moe_kernel,
        grid=(_E,),
        in_specs=[
            pl.BlockSpec((_T, _D), lambda e: (0, 0)),
            pl.BlockSpec((_E, _D), lambda e: (0, 0)),
            pl.BlockSpec((1, _E), lambda e: (0, 0)),
            pl.BlockSpec((1, _D, _F), lambda e: (e, 0, 0)),
            pl.BlockSpec((1, _D, _F), lambda e: (e, 0, 0)),
            pl.BlockSpec((1, _F, _D), lambda e: (e, 0, 0)),
        ],
        out_specs=pl.BlockSpec((_T, _D), lambda e: (0, 0)),
        out_shape=jax.ShapeDtypeStruct((_T, _D), jnp.float32),
        scratch_shapes=[pltpu.VMEM((_T, _E), jnp.float32)],
        compiler_params=pltpu.CompilerParams(
            dimension_semantics=("arbitrary",),
        ),
    )(tt_input, gate_w, gb2, w_gate, w_up, w_down)


# speedup vs baseline: 1.7965x; 1.0506x over previous
import jax
import jax.numpy as jnp
from jax.experimental import pallas as pl
from jax.experimental.pallas import tpu as pltpu

_E = 64
_K = 8
_NG = 8
_KG = 4
_GS = _E // _NG
_D = 1024
_F = 512
_T = 128
_SCALE = 2.5
_EPB = 2


def _gate_weights(x, gw, gb):
    logits = jax.lax.dot_general(x, gw, (((1,), (1,)), ((), ())),
                                 preferred_element_type=jnp.float32)
    scores = jax.nn.sigmoid(logits)
    s4c = scores + gb
    lane = jax.lax.broadcasted_iota(jnp.int32, (_T, _E), 1)
    grp = lane // _GS
    neg = jnp.float32(-jnp.inf)

    gs_cols = []
    for g in range(_NG):
        vg = jnp.where(grp == g, s4c, neg)
        m1 = jnp.max(vg, axis=1, keepdims=True)
        j1 = jnp.min(jnp.where(vg == m1, lane, _E), axis=1, keepdims=True)
        m2 = jnp.max(jnp.where(lane == j1, neg, vg), axis=1, keepdims=True)
        gs_cols.append(m1 + m2)
    gs = jnp.concatenate(gs_cols, axis=1)

    giota = jax.lax.broadcasted_iota(jnp.int32, (_T, _NG), 1)
    gmask = jnp.zeros((_T, _NG), jnp.bool_)
    for _ in range(_KG):
        m = jnp.max(gs, axis=1, keepdims=True)
        jm = jnp.min(jnp.where(gs == m, giota, _NG), axis=1, keepdims=True)
        pick = giota == jm
        gmask = jnp.logical_or(gmask, pick)
        gs = jnp.where(pick, neg, gs)

    emask = jnp.zeros((_T, _E), jnp.bool_)
    for g in range(_NG):
        emask = jnp.logical_or(emask, jnp.logical_and(grp == g, gmask[:, g:g + 1]))

    masked = jnp.where(emask, s4c, neg)
    sel = jnp.zeros((_T, _E), jnp.bool_)
    for _ in range(_K):
        m = jnp.max(masked, axis=1, keepdims=True)
        jm = jnp.min(jnp.where(masked == m, lane, _E), axis=1, keepdims=True)
        pick = lane == jm
        sel = jnp.logical_or(sel, pick)
        masked = jnp.where(pick, neg, masked)

    w = jnp.where(sel, scores, 0.0)
    denom = jnp.sum(w, axis=1, keepdims=True) + 1e-20
    return w * (_SCALE / denom)


def _moe_kernel(x_ref, gw_ref, gb_ref, wg_ref, wu_ref, wd_ref, out_ref, wd_scr):
    e = pl.program_id(0)

    @pl.when(e == 0)
    def _():
        wd_scr[...] = _gate_weights(x_ref[...], gw_ref[...], gb_ref[...])

    x = x_ref[...]
    lane = jax.lax.broadcasted_iota(jnp.int32, (_T, _E), 1)
    contrib = jnp.zeros((_T, _D), jnp.float32)
    for j in range(_EPB):
        h = jnp.dot(x, wg_ref[j], preferred_element_type=jnp.float32)
        u = jnp.dot(x, wu_ref[j], preferred_element_type=jnp.float32)
        act = (h * jax.nn.sigmoid(h)) * u
        y = jnp.dot(act, wd_ref[j], preferred_element_type=jnp.float32)
        wcol = jnp.sum(jnp.where(lane == _EPB * e + j, wd_scr[...], 0.0),
                       axis=1, keepdims=True)
        contrib = contrib + y * wcol

    @pl.when(e == 0)
    def _():
        out_ref[...] = contrib

    @pl.when(e > 0)
    def _():
        out_ref[...] += contrib


def kernel(tt_input, gate_w, gate_bias, w_gate, w_up, w_down):
    gb2 = gate_bias.reshape(1, _E)
    return pl.pallas_call(
        _moe_kernel,
        grid=(_E // _EPB,),
        in_specs=[
            pl.BlockSpec((_T, _D), lambda e: (0, 0)),
            pl.BlockSpec((_E, _D), lambda e: (0, 0)),
            pl.BlockSpec((1, _E), lambda e: (0, 0)),
            pl.BlockSpec((_EPB, _D, _F), lambda e: (e, 0, 0)),
            pl.BlockSpec((_EPB, _D, _F), lambda e: (e, 0, 0)),
            pl.BlockSpec((_EPB, _F, _D), lambda e: (e, 0, 0)),
        ],
        out_specs=pl.BlockSpec((_T, _D), lambda e: (0, 0)),
        out_shape=jax.ShapeDtypeStruct((_T, _D), jnp.float32),
        scratch_shapes=[pltpu.VMEM((_T, _E), jnp.float32)],
        compiler_params=pltpu.CompilerParams(
            dimension_semantics=("arbitrary",),
        ),
    )(tt_input, gate_w, gb2, w_gate, w_up, w_down)
